# baseline (speedup 1.0000x reference)
import functools

import jax
import jax.numpy as jnp
from jax import lax
from jax.experimental import pallas as pl
from jax.experimental.pallas import tpu as pltpu
from jax.experimental.pallas import tpu_sc as plsc

LAT, LON, NQ = 1801, 3600, 1048576
NC, NS, L = 2, 16, 16
NW = NC * NS
QPW = NQ // NW
CH = 2048
NCH = QPW // CH

NV = LAT * LON
RSTEP = LON // 8
NE = 1808

_BR = 64
_NA = 29
_T2LAT = 2 * _NA * _BR
SB = _NA * _BR * LON // 8
T2ROWS = _T2LAT * LON // 8


def _t2_body(xa_ref, xb_ref, o_ref):
    g = pl.program_id(0)
    xa = xa_ref[...]
    xb = xb_ref[...]
    nxt = jnp.concatenate([xa[1:, :], xb[:1, :]], axis=0)
    shifted = jnp.concatenate([xa[:, 4:], nxt[:, :4]], axis=1)
    o_ref[...] = jnp.where(g < _NA, xa, shifted)


def _build_t2(values):
    nb8 = LAT // 8
    return pl.pallas_call(
        _t2_body,
        grid=(2 * _NA,),
        in_specs=[
            pl.BlockSpec((_BR, LON),
                         lambda g: (jnp.where(g < _NA, g, g - _NA), 0)),
            pl.BlockSpec((8, LON),
                         lambda g: (jnp.clip(8 * (g - _NA + 1), 0, nb8), 0)),
        ],
        out_specs=pl.BlockSpec((_BR, LON), lambda g: (g, 0)),
        out_shape=jax.ShapeDtypeStruct((_T2LAT, LON), jnp.float32),
    )(values, values)


def _buf_set():
    return (
        [pltpu.VMEM((CH,), jnp.float32)] * 2
        + [pltpu.VMEM((CH,), jnp.int32)] * 5
        + [pltpu.VMEM((CH,), jnp.float32)] * 2
        + [pltpu.VMEM((CH, 8), jnp.float32)] * 2
    )


def _make_interp():
    mesh = plsc.VectorSubcoreMesh(core_axis_name="c", subcore_axis_name="s")

    @functools.partial(
        pl.kernel,
        out_type=jax.ShapeDtypeStruct((NQ,), jnp.float32),
        mesh=mesh,
        compiler_params=pltpu.CompilerParams(
            needs_layout_passes=False, use_tc_tiling_on_sc=False),
        scratch_types=[
            _buf_set(),
            _buf_set(),
            pltpu.VMEM((CH,), jnp.float32),
            pltpu.VMEM((NE,), jnp.int32),
            pltpu.VMEM((NE, 8), jnp.float32),
            pltpu.SemaphoreType.DMA,
            pltpu.SemaphoreType.DMA,
        ],
    )
    def interp(t2, qlat, qlon, out, bufs_a, bufs_b, res_v, eidx_v, e8_v,
               sem_a, sem_b):
        wid = lax.axis_index("s") * NC + lax.axis_index("c")
        base = wid * QPW

        @pl.loop(0, NE // L)
        def _eidx(kv):
            eidx_v[pl.ds(kv * L, L)] = jnp.minimum(
                (lax.iota(jnp.int32, L) + kv * L) * RSTEP, (LAT - 1) * RSTEP)

        pltpu.async_copy(t2.at[eidx_v], e8_v, sem_a).wait()

        def prep(c, bufs):
            xq_v, yq_v, r0_v, r1_v, cb_v, ix_v, wr_v, t_v, u_v = bufs[:9]
            off = base + c * CH
            pltpu.sync_copy(qlat.at[pl.ds(off, CH)], xq_v)
            pltpu.sync_copy(qlon.at[pl.ds(off, CH)], yq_v)

            @pl.loop(0, CH // L, unroll=4)
            def _idx(kv):
                s = pl.ds(kv * L, L)
                fx = (xq_v[s] + 90.0) * 10.0
                fy = (yq_v[s] + 180.0) * 10.0
                ix = jnp.minimum(fx.astype(jnp.int32), LAT - 2)
                jy = jnp.minimum(fy.astype(jnp.int32), LON - 1)
                t_v[s] = fx - ix.astype(jnp.float32)
                u_v[s] = fy - jy.astype(jnp.float32)
                o = jy & 7
                shifted = o == 7
                r0 = ((ix * LON + jy) >> 3) + jnp.where(shifted, SB, 0)
                r0_v[s] = r0
                r1_v[s] = r0 + RSTEP
                cb_v[s] = jnp.where(shifted, 3, o)
                ix_v[s] = ix
                wr_v[s] = jnp.where(jy == LON - 1, 1, 0)

        def copies(bufs, sem):
            r0_v, r1_v = bufs[2], bufs[3]
            g0_v, g1_v = bufs[9], bufs[10]
            return (
                pltpu.make_async_copy(t2.at[r0_v], g0_v, sem),
                pltpu.make_async_copy(t2.at[r1_v], g1_v, sem),
            )

        def fire(bufs, sem):
            for cp in copies(bufs, sem):
                cp.start()

        def drain(c, bufs, sem):
            for cp in copies(bufs, sem):
                cp.wait()
            cb_v, ix_v, wr_v, t_v, u_v = bufs[4:9]
            g0_v, g1_v = bufs[9], bufs[10]

            @pl.loop(0, CH // L, unroll=4)
            def _blend(kv):
                s = pl.ds(kv * L, L)
                q = lax.iota(jnp.int32, L) + kv * L
                zero = jnp.zeros((L,), jnp.int32)
                cb = cb_v[s]
                wrap = wr_v[s] > 0
                ix = ix_v[s]
                v00 = plsc.load_gather(g0_v, [q, cb])
                v10 = plsc.load_gather(g1_v, [q, cb])
                v01 = jnp.where(wrap,
                                plsc.load_gather(e8_v, [ix, zero]),
                                plsc.load_gather(g0_v, [q, cb + 1]))
                v11 = jnp.where(wrap,
                                plsc.load_gather(e8_v, [ix + 1, zero]),
                                plsc.load_gather(g1_v, [q, cb + 1]))
                t = t_v[s]
                u = u_v[s]
                res_v[s] = ((1.0 - t) * (1.0 - u) * v00
                            + (1.0 - t) * u * v01
                            + t * (1.0 - u) * v10
                            + t * u * v11)

            pltpu.sync_copy(res_v, out.at[pl.ds(base + c * CH, CH)])

        prep(0, bufs_a)
        fire(bufs_a, sem_a)

        @pl.loop(0, NCH // 2)
        def _steady(h):
            c0 = 2 * h
            prep(c0 + 1, bufs_b)
            fire(bufs_b, sem_b)
            drain(c0, bufs_a, sem_a)

            @pl.when(h < NCH // 2 - 1)
            def _refill():
                prep(c0 + 2, bufs_a)
                fire(bufs_a, sem_a)

            drain(c0 + 1, bufs_b, sem_b)

    return interp


_interp = _make_interp()


def kernel(values, grid_latitude, grid_longitude, query_latitude, query_longitude):
    del grid_latitude, grid_longitude
    t2 = _build_t2(values).reshape(T2ROWS, 8)
    return _interp(t2, query_latitude, query_longitude)


# speedup vs baseline: 1.0763x; 1.0763x over previous
import functools

import jax
import jax.numpy as jnp
from jax import lax
from jax.experimental import pallas as pl
from jax.experimental.pallas import tpu as pltpu
from jax.experimental.pallas import tpu_sc as plsc

LAT, LON, NQ = 1801, 3600, 1048576
NC, NS, L = 2, 16, 16
NW = NC * NS
QPW = NQ // NW
CH = 4096
NCH = QPW // CH


def _buf_set():
    return (
        [pltpu.VMEM((CH,), jnp.float32)] * 2
        + [pltpu.VMEM((CH,), jnp.int32)] * 4
        + [pltpu.VMEM((CH,), jnp.float32)] * 2
        + [pltpu.VMEM((CH,), jnp.float32)] * 4
    )


def _make_interp():
    mesh = plsc.VectorSubcoreMesh(core_axis_name="c", subcore_axis_name="s")

    @functools.partial(
        pl.kernel,
        out_type=jax.ShapeDtypeStruct((NQ,), jnp.float32),
        mesh=mesh,
        scratch_types=[
            _buf_set(),
            _buf_set(),
            pltpu.VMEM((CH,), jnp.float32),
            pltpu.SemaphoreType.DMA,
            pltpu.SemaphoreType.DMA,
        ],
    )
    def interp(values, qlat, qlon, out, bufs_a, bufs_b, res_v, sem_a, sem_b):
        wid = lax.axis_index("s") * NC + lax.axis_index("c")
        base = wid * QPW

        def prep(c, bufs):
            xq_v, yq_v, i00_v, i01_v, i10_v, i11_v, t_v, u_v = bufs[:8]
            off = base + c * CH
            pltpu.sync_copy(qlat.at[pl.ds(off, CH)], xq_v)
            pltpu.sync_copy(qlon.at[pl.ds(off, CH)], yq_v)

            @pl.loop(0, CH // L, unroll=4)
            def _idx(kv):
                s = pl.ds(kv * L, L)
                fx = (xq_v[s] + 90.0) * 10.0
                fy = (yq_v[s] + 180.0) * 10.0
                ix = jnp.minimum(fx.astype(jnp.int32), LAT - 2)
                jy = jnp.minimum(fy.astype(jnp.int32), LON - 1)
                t_v[s] = fx - ix.astype(jnp.float32)
                u_v[s] = fy - jy.astype(jnp.float32)
                f00 = ix * LON + jy
                f01 = jnp.where(jy == LON - 1, ix * LON, f00 + 1)
                i00_v[s] = f00
                i01_v[s] = f01
                i10_v[s] = f00 + LON
                i11_v[s] = f01 + LON

        def copies(bufs, sem):
            i00_v, i01_v, i10_v, i11_v = bufs[2:6]
            g00_v, g01_v, g10_v, g11_v = bufs[8:12]
            return (
                pltpu.make_async_copy(values.at[i00_v], g00_v, sem),
                pltpu.make_async_copy(values.at[i01_v], g01_v, sem),
                pltpu.make_async_copy(values.at[i10_v], g10_v, sem),
                pltpu.make_async_copy(values.at[i11_v], g11_v, sem),
            )

        def fire(bufs, sem):
            for cp in copies(bufs, sem):
                cp.start()

        def drain(c, bufs, sem):
            for cp in copies(bufs, sem):
                cp.wait()
            t_v, u_v = bufs[6:8]
            g00_v, g01_v, g10_v, g11_v = bufs[8:12]

            @pl.loop(0, CH // L, unroll=4)
            def _blend(kv):
                s = pl.ds(kv * L, L)
                t = t_v[s]
                u = u_v[s]
                res_v[s] = ((1.0 - t) * (1.0 - u) * g00_v[s]
                            + (1.0 - t) * u * g01_v[s]
                            + t * (1.0 - u) * g10_v[s]
                            + t * u * g11_v[s])

            pltpu.sync_copy(res_v, out.at[pl.ds(base + c * CH, CH)])

        prep(0, bufs_a)
        fire(bufs_a, sem_a)

        @pl.loop(0, NCH // 2)
        def _steady(h):
            c0 = 2 * h
            prep(c0 + 1, bufs_b)
            fire(bufs_b, sem_b)
            drain(c0, bufs_a, sem_a)

            @pl.when(h < NCH // 2 - 1)
            def _refill():
                prep(c0 + 2, bufs_a)
                fire(bufs_a, sem_a)

            drain(c0 + 1, bufs_b, sem_b)

    return interp


_interp = _make_interp()


def kernel(values, grid_latitude, grid_longitude, query_latitude, query_longitude):
    del grid_latitude, grid_longitude
    return _interp(values.reshape(LAT * LON), query_latitude, query_longitude)
